# baseline (speedup 1.0000x reference)
import functools

import jax
import jax.numpy as jnp
from jax import lax
from jax.experimental import pallas as pl
from jax.experimental.pallas import tpu as pltpu
from jax.experimental.pallas import tpu_sc as plsc

_LOG1E4 = 9.210340371976184
_INV2PI = 0.15915494309189535

_C0 = 6.283088507310033
_C1 = -41.333250612374165
_C2 = 81.40014502793045
_C3 = -74.67624173688598
_C4 = 33.16885008474881

_NW = 32
_CP = 8


def _sc_body(x_hbm, o_hbm, inv_v, ph_v, xbuf, obuf, *, b, s, e):
    cid = lax.axis_index("c")
    sid = lax.axis_index("s")
    wid = sid * 2 + cid

    nj = e // 16

    def fill(j, carry):
        ei = lax.iota(jnp.int32, 16) + j * 16
        ef = ei.astype(jnp.float32)
        expo = (ef - jnp.mod(ef, 2.0)) * (1.0 / e)
        inv_v[pl.ds(j * 16, 16)] = jnp.exp(-_LOG1E4 * expo) * _INV2PI
        ph_v[pl.ds(j * 16, 16)] = jnp.where(ei % 2 == 0, 0.0, 0.25)
        return carry

    lax.fori_loop(0, nj, fill, 0)

    npos = s // _NW
    p0w = wid * npos
    nchunks = npos // _CP

    def chunk_body(c, carry):
        pos0 = p0w + c * _CP
        for bb in range(b):
            pltpu.sync_copy(x_hbm.at[bb, pl.ds(pos0, _CP)], xbuf.at[bb])

        for bb in range(b):
            pltpu.sync_copy(xbuf.at[bb], o_hbm.at[bb, pl.ds(pos0, _CP)])
        return carry

    lax.fori_loop(0, nchunks, chunk_body, 0)


def kernel(x):
    B, S, E = x.shape
    run = pl.kernel(
        functools.partial(_sc_body, b=B, s=S, e=E),
        out_type=jax.ShapeDtypeStruct((B, S, E), jnp.float32),
        mesh=plsc.VectorSubcoreMesh(core_axis_name="c", subcore_axis_name="s"),
        scratch_types=[
            pltpu.VMEM((E,), jnp.float32),
            pltpu.VMEM((E,), jnp.float32),
            pltpu.VMEM((B, _CP, E), jnp.float32),
            pltpu.VMEM((B, _CP, E), jnp.float32),
        ],
    )
    return run(x)


# speedup vs baseline: 1.2646x; 1.2646x over previous
import functools

import jax
import jax.numpy as jnp
from jax import lax
from jax.experimental import pallas as pl
from jax.experimental.pallas import tpu as pltpu
from jax.experimental.pallas import tpu_sc as plsc

_LOG1E4 = 9.210340371976184
_INV2PI = 0.15915494309189535

_C0 = 6.283088507310033
_C1 = -41.333250612374165
_C2 = 81.40014502793045
_C3 = -74.67624173688598
_C4 = 33.16885008474881

_NW = 32
_CP = 4


def _sc_body(x_hbm, o_hbm, inv_v, ph_v, xbuf, obuf, insem, outsem,
             *, b, s, e):
    cid = lax.axis_index("c")
    sid = lax.axis_index("s")
    wid = sid * 2 + cid

    nj = e // 16

    def fill(j, carry):
        ei = lax.iota(jnp.int32, 16) + j * 16
        ef = ei.astype(jnp.float32)
        expo = (ef - jnp.mod(ef, 2.0)) * (1.0 / e)
        inv_v[pl.ds(j * 16, 16)] = jnp.exp(-_LOG1E4 * expo) * _INV2PI
        ph_v[pl.ds(j * 16, 16)] = jnp.where(ei % 2 == 0, 0.0, 0.25)
        return carry

    lax.fori_loop(0, nj, fill, 0)

    npos = s // _NW
    p0w = wid * npos
    nchunks = npos // _CP

    def in_copies(slot, c):
        pos0 = p0w + c * _CP
        return [
            pltpu.make_async_copy(x_hbm.at[bb, pl.ds(pos0, _CP)],
                                  xbuf.at[slot, bb], insem)
            for bb in range(b)
        ]

    def out_copies(slot, c):
        pos0 = p0w + c * _CP
        return [
            pltpu.make_async_copy(obuf.at[slot, bb],
                                  o_hbm.at[bb, pl.ds(pos0, _CP)], outsem)
            for bb in range(b)
        ]

    def compute(slot, c):
        pos0 = p0w + c * _CP

        def jbody(j, jcarry):
            inv = inv_v[pl.ds(j * 16, 16)]
            ph = ph_v[pl.ds(j * 16, 16)]
            for p in range(_CP):
                posf = (pos0 + p).astype(jnp.float32)
                y = posf * inv + ph
                f = (y + 0.5).astype(jnp.int32).astype(jnp.float32)
                r = y - f
                r2 = r * r
                pp = _C3 + r2 * _C4
                pp = _C2 + r2 * pp
                pp = _C1 + r2 * pp
                pp = _C0 + r2 * pp
                enc = r * pp
                for bb in range(b):
                    xv = xbuf[slot, bb, p, pl.ds(j * 16, 16)]
                    obuf[slot, bb, p, pl.ds(j * 16, 16)] = jnp.where(
                        xv == 0.0, 0.0, enc)
            return jcarry

        lax.fori_loop(0, nj, jbody, 0)

    for cp in in_copies(0, 0):
        cp.start()

    def step(t, carry):
        for slot in (0, 1):
            c = 2 * t + slot

            @pl.when(c + 1 < nchunks)
            def _():
                for cp in in_copies(slot ^ 1, c + 1):
                    cp.start()

            for cp in in_copies(slot, c):
                cp.wait()

            @pl.when(c > 1)
            def _():
                for cp in out_copies(slot, c - 2):
                    cp.wait()

            compute(slot, c)
            for cp in out_copies(slot, c):
                cp.start()
        return carry

    lax.fori_loop(0, nchunks // 2, step, 0)
    for cp in out_copies(0, nchunks - 2):
        cp.wait()
    for cp in out_copies(1, nchunks - 1):
        cp.wait()


def kernel(x):
    B, S, E = x.shape
    run = pl.kernel(
        functools.partial(_sc_body, b=B, s=S, e=E),
        out_type=jax.ShapeDtypeStruct((B, S, E), jnp.float32),
        mesh=plsc.VectorSubcoreMesh(core_axis_name="c", subcore_axis_name="s"),
        scratch_types=[
            pltpu.VMEM((E,), jnp.float32),
            pltpu.VMEM((E,), jnp.float32),
            pltpu.VMEM((2, B, _CP, E), jnp.float32),
            pltpu.VMEM((2, B, _CP, E), jnp.float32),
            pltpu.SemaphoreType.DMA,
            pltpu.SemaphoreType.DMA,
        ],
    )
    return run(x)


# speedup vs baseline: 1.8957x; 1.4991x over previous
import functools

import jax
import jax.numpy as jnp
from jax.experimental import pallas as pl
from jax.experimental.pallas import tpu as pltpu

_LOG1E4 = 9.210340371976184
_INV2PI = 0.15915494309189535

_C0 = 6.2797307080712255
_C1 = -41.13626070861352
_C2 = 78.32711789390086
_C3 = -57.11617448291767


def _pos_emb_kernel(x_ref, o_ref, inv_ref, ph_ref, *, ts: int, e: int,
                    s: int):
    i = pl.program_id(0)

    @pl.when(i == 0)
    def _():
        ei = jax.lax.broadcasted_iota(jnp.int32, (1, e), 1)
        ef = ei.astype(jnp.float32)
        expo = (ef - jnp.mod(ef, 2.0)) * (1.0 / e)
        inv_ref[...] = jnp.exp(-_LOG1E4 * expo) * _INV2PI
        ph_ref[...] = jnp.where(ei % 2 == 0, 0.0, 0.25)

    base = (i * ts) % s
    pos = (base + jax.lax.broadcasted_iota(jnp.int32, (ts, 1), 0)).astype(
        jnp.float32)
    y = pos * inv_ref[...] + ph_ref[...]
    r = y - jnp.floor(y + 0.5)
    r2 = r * r
    p = _C2 + r2 * _C3
    p = _C1 + r2 * p
    p = _C0 + r2 * p
    enc = r * p
    xv = x_ref[...]
    o_ref[...] = jnp.where(xv == 0.0, 0.0, enc)


def kernel(x):
    B, S, E = x.shape
    TS = 512
    xf = x.reshape(B * S, E)
    out = pl.pallas_call(
        functools.partial(_pos_emb_kernel, ts=TS, e=E, s=S),
        grid=(B * S // TS,),
        in_specs=[pl.BlockSpec((TS, E), lambda i: (i, 0))],
        out_specs=pl.BlockSpec((TS, E), lambda i: (i, 0)),
        out_shape=jax.ShapeDtypeStruct((B * S, E), jnp.float32),
        scratch_shapes=[
            pltpu.VMEM((1, E), jnp.float32),
            pltpu.VMEM((1, E), jnp.float32),
        ],
    )(xf)
    return out.reshape(B, S, E)


# speedup vs baseline: 2.5568x; 1.3487x over previous
import functools

import jax
import jax.numpy as jnp
from jax.experimental import pallas as pl
from jax.experimental.pallas import tpu as pltpu

_LOG1E4 = 9.210340371976184
_INV2PI = 0.15915494309189535

_C0 = 6.2797307080712255
_C1 = -41.13626070861352
_C2 = 78.32711789390086
_C3 = -57.11617448291767


def _pos_emb_kernel(x_ref, o_ref, inv_ref, ph_ref, *, ts: int, e: int):
    i = pl.program_id(0)

    @pl.when(i == 0)
    def _():
        ei = jax.lax.broadcasted_iota(jnp.int32, (1, e), 1)
        ef = ei.astype(jnp.float32)
        expo = (ef - jnp.mod(ef, 2.0)) * (1.0 / e)
        inv_ref[...] = jnp.exp(-_LOG1E4 * expo) * _INV2PI
        ph_ref[...] = jnp.where(ei % 2 == 0, 0.0, 0.25)

    pos = (i * ts + jax.lax.broadcasted_iota(jnp.int32, (ts, 1), 0)).astype(
        jnp.float32)
    y = pos * inv_ref[...] + ph_ref[...]
    r = y - jnp.floor(y + 0.5)
    r2 = r * r
    p = _C2 + r2 * _C3
    p = _C1 + r2 * p
    p = _C0 + r2 * p
    enc = r * p
    xv = x_ref[...]
    o_ref[...] = jnp.where(xv == 0.0, 0.0, enc[None, :, :])


def kernel(x):
    B, S, E = x.shape
    TS = 512
    grid = (S // TS,)
    return pl.pallas_call(
        functools.partial(_pos_emb_kernel, ts=TS, e=E),
        grid=grid,
        in_specs=[pl.BlockSpec((B, TS, E), lambda i: (0, i, 0))],
        out_specs=pl.BlockSpec((B, TS, E), lambda i: (0, i, 0)),
        out_shape=jax.ShapeDtypeStruct((B, S, E), jnp.float32),
        scratch_shapes=[
            pltpu.VMEM((1, E), jnp.float32),
            pltpu.VMEM((1, E), jnp.float32),
        ],
    )(x)
